# baseline (speedup 1.0000x reference)
import functools
import numpy as np
import jax
import jax.numpy as jnp
from jax import lax
from jax.experimental import pallas as pl
from jax.experimental.pallas import tpu as pltpu
from jax.experimental.pallas import tpu_sc as plsc

_C = 26
_D = 32
_B = 16384
_CARD = 100000
_NW = 32
_TOTAL = _B * _C
_PER_W = _TOTAL // _NW
_CHUNK = 832
_NCHUNK = _PER_W // _CHUNK
_L = 16

_NROWS = _C * _CARD + 1
_CB = 2048
_GRID = (_NROWS + _CB - 1) // _CB
_ROWS_PAD = _GRID * _CB

_OFF_EXP = np.asarray(
    [(p % _C) * _CARD for p in range(208)], dtype=np.int32)


def _body(x_ref, tab_ref, bias_ref, off_ref, out_ref,
          idx_v, off_v, bias_v, rows0, rows1,
          gsem0, gsem1, osem0, osem1):
  cid = lax.axis_index("c")
  sid = lax.axis_index("s")
  wid = sid * 2 + cid
  base = wid * _PER_W

  pltpu.sync_copy(x_ref.at[pl.ds(base, _PER_W)], idx_v)
  pltpu.sync_copy(bias_ref, bias_v)
  pltpu.sync_copy(off_ref, off_v)

  def offs_body(k, carry):
    s = k * _L
    o = off_v[pl.ds(lax.rem(k, 13) * _L, _L)]
    idx_v[pl.ds(s, _L)] = idx_v[pl.ds(s, _L)] + o
    return carry
  lax.fori_loop(0, _PER_W // _L, offs_body, 0)

  rows_bufs = (rows0, rows1)
  gsems = (gsem0, gsem1)
  osems = (osem0, osem1)
  ghandles = [None, None]
  ohandles = [None, None]

  def start_gather(g):
    cur = g % 2
    ghandles[cur] = pltpu.async_copy(
        tab_ref.at[idx_v.at[pl.ds(g * _CHUNK, _CHUNK)]],
        rows_bufs[cur], gsems[cur])

  def bias_add(rv):
    def c_body(c, carry):
      b0 = bias_v[c, pl.ds(0, _L)]
      b1 = bias_v[c, pl.ds(_L, _L)]
      def m_body(m, carry2):
        r = m * _C + c
        rv[r, pl.ds(0, _L)] = rv[r, pl.ds(0, _L)] + b0
        rv[r, pl.ds(_L, _L)] = rv[r, pl.ds(_L, _L)] + b1
        return carry2
      return lax.fori_loop(0, _CHUNK // _C, m_body, carry)
    lax.fori_loop(0, _C, c_body, 0)

  start_gather(0)
  for g in range(_NCHUNK):
    cur = g % 2
    nxt = 1 - cur
    if g + 1 < _NCHUNK:
      if ohandles[nxt] is not None:
        ohandles[nxt].wait()
        ohandles[nxt] = None
      start_gather(g + 1)
    ghandles[cur].wait()
    bias_add(rows_bufs[cur])
    ohandles[cur] = pltpu.async_copy(
        rows_bufs[cur],
        out_ref.at[pl.ds(base + g * _CHUNK, _CHUNK)],
        osems[cur])
  for h in ohandles:
    if h is not None:
      h.wait()


@functools.partial(
    pl.kernel,
    out_type=jax.ShapeDtypeStruct((_TOTAL, _D), jnp.float32),
    mesh=plsc.VectorSubcoreMesh(core_axis_name="c", subcore_axis_name="s"),
    compiler_params=pltpu.CompilerParams(use_tc_tiling_on_sc=False),
    scratch_types=[
        pltpu.VMEM((_PER_W,), jnp.int32),
        pltpu.VMEM((len(_OFF_EXP),), jnp.int32),
        pltpu.VMEM((_C, _D), jnp.float32),
        pltpu.VMEM((_CHUNK, _D), jnp.float32),
        pltpu.VMEM((_CHUNK, _D), jnp.float32),
        pltpu.SemaphoreType.DMA,
        pltpu.SemaphoreType.DMA,
        pltpu.SemaphoreType.DMA,
        pltpu.SemaphoreType.DMA,
    ],
)
def _sc_lookup(x_ref, tab_ref, bias_ref, off_ref, out_ref, *scratch):
  _body(x_ref, tab_ref, bias_ref, off_ref, out_ref, *scratch)


def _convert_body(x_ref, o_ref):
  x = x_ref[...]
  t = jnp.transpose(jnp.reshape(x, (_D, _CB // 4, 4)), (1, 2, 0))
  o_ref[...] = jnp.reshape(t, (_CB // 4, 4 * _D))


def _tc_convert(tab_t):
  return pl.pallas_call(
      _convert_body,
      grid=(_GRID,),
      in_specs=[pl.BlockSpec((_D, _CB), lambda g: (0, g))],
      out_specs=pl.BlockSpec((_CB // 4, 4 * _D), lambda g: (g, 0)),
      out_shape=jax.ShapeDtypeStruct((_GRID * _CB // 4, 4 * _D), jnp.float32),
  )(tab_t)


def kernel(x_cat, table, bias):
  x_flat = x_cat.astype(jnp.int32).reshape(-1)
  tab_t = jnp.transpose(table)
  conv = _tc_convert(tab_t)
  table_lin = conv.reshape(-1).reshape(_ROWS_PAD, _D)
  out = _sc_lookup(x_flat, table_lin, bias, jnp.asarray(_OFF_EXP))
  return out.reshape(_B, _C, _D)


# speedup vs baseline: 5.4838x; 5.4838x over previous
import functools
import numpy as np
import jax
import jax.numpy as jnp
from jax import lax
from jax.experimental import pallas as pl
from jax.experimental.pallas import tpu as pltpu
from jax.experimental.pallas import tpu_sc as plsc

_C = 26
_D = 32
_B = 16384
_CARD = 100000
_NW = 32
_TOTAL = _B * _C
_PER_W = _TOTAL // _NW
_CHUNK = 832
_NCHUNK = _PER_W // _CHUNK
_L = 16

_NROWS = _C * _CARD + 1
_BJ = 1024
_STRIPE = 650240
_GRID = _STRIPE // _BJ
_ROWS_PAD = 4 * _STRIPE

_OFF_EXP = np.asarray(
    [(p % _C) * _CARD for p in range(208)], dtype=np.int32)


def _body(x_ref, tab_ref, bias_ref, off_ref, out_ref,
          idx_v, off_v, bias_v, rows0, rows1,
          gsem0, gsem1, osem0, osem1):
  cid = lax.axis_index("c")
  sid = lax.axis_index("s")
  wid = sid * 2 + cid
  base = wid * _PER_W

  pltpu.sync_copy(x_ref.at[pl.ds(base, _PER_W)], idx_v)
  pltpu.sync_copy(bias_ref, bias_v)
  pltpu.sync_copy(off_ref, off_v)

  def offs_body(k, carry):
    s = k * _L
    o = off_v[pl.ds(lax.rem(k, 13) * _L, _L)]
    r = idx_v[pl.ds(s, _L)] + o
    q = lax.div(r, _STRIPE)
    idx_v[pl.ds(s, _L)] = 4 * r - q * (4 * _STRIPE - 1)
    return carry
  lax.fori_loop(0, _PER_W // _L, offs_body, 0)

  rows_bufs = (rows0, rows1)
  gsems = (gsem0, gsem1)
  osems = (osem0, osem1)
  ghandles = [None, None]
  ohandles = [None, None]

  def start_gather(g):
    cur = g % 2
    ghandles[cur] = pltpu.async_copy(
        tab_ref.at[idx_v.at[pl.ds(g * _CHUNK, _CHUNK)]],
        rows_bufs[cur], gsems[cur])

  def bias_add(rv):
    def c_body(c, carry):
      b0 = bias_v[c, pl.ds(0, _L)]
      b1 = bias_v[c, pl.ds(_L, _L)]
      def m_body(m, carry2):
        r = m * _C + c
        rv[r, pl.ds(0, _L)] = rv[r, pl.ds(0, _L)] + b0
        rv[r, pl.ds(_L, _L)] = rv[r, pl.ds(_L, _L)] + b1
        return carry2
      return lax.fori_loop(0, _CHUNK // _C, m_body, carry)
    lax.fori_loop(0, _C, c_body, 0)

  start_gather(0)
  for g in range(_NCHUNK):
    cur = g % 2
    nxt = 1 - cur
    if g + 1 < _NCHUNK:
      if ohandles[nxt] is not None:
        ohandles[nxt].wait()
        ohandles[nxt] = None
      start_gather(g + 1)
    ghandles[cur].wait()
    bias_add(rows_bufs[cur])
    ohandles[cur] = pltpu.async_copy(
        rows_bufs[cur],
        out_ref.at[pl.ds(base + g * _CHUNK, _CHUNK)],
        osems[cur])
  for h in ohandles:
    if h is not None:
      h.wait()


@functools.partial(
    pl.kernel,
    out_type=jax.ShapeDtypeStruct((_TOTAL, _D), jnp.float32),
    mesh=plsc.VectorSubcoreMesh(core_axis_name="c", subcore_axis_name="s"),
    compiler_params=pltpu.CompilerParams(use_tc_tiling_on_sc=False),
    scratch_types=[
        pltpu.VMEM((_PER_W,), jnp.int32),
        pltpu.VMEM((len(_OFF_EXP),), jnp.int32),
        pltpu.VMEM((_C, _D), jnp.float32),
        pltpu.VMEM((_CHUNK, _D), jnp.float32),
        pltpu.VMEM((_CHUNK, _D), jnp.float32),
        pltpu.SemaphoreType.DMA,
        pltpu.SemaphoreType.DMA,
        pltpu.SemaphoreType.DMA,
        pltpu.SemaphoreType.DMA,
    ],
)
def _sc_lookup(x_ref, tab_ref, bias_ref, off_ref, out_ref, *scratch):
  _body(x_ref, tab_ref, bias_ref, off_ref, out_ref, *scratch)


def _convert_body(x0, x1, x2, x3, o_ref):
  parts = [jnp.transpose(x[...], (1, 0)) for x in (x0, x1, x2, x3)]
  o_ref[...] = jnp.concatenate(parts, axis=1)


def _tc_convert(tab_t):
  return pl.pallas_call(
      _convert_body,
      grid=(_GRID,),
      in_specs=[
          pl.BlockSpec((_D, _BJ), (lambda g, k=k: (0, k * _GRID + g)))
          for k in range(4)
      ],
      out_specs=pl.BlockSpec((_BJ, 4 * _D), lambda g: (g, 0)),
      out_shape=jax.ShapeDtypeStruct((_STRIPE, 4 * _D), jnp.float32),
  )(tab_t, tab_t, tab_t, tab_t)


def kernel(x_cat, table, bias):
  x_flat = x_cat.astype(jnp.int32).reshape(-1)
  tab_t = jnp.transpose(table)
  conv = _tc_convert(tab_t)
  table_lin = conv.reshape(-1).reshape(_ROWS_PAD, _D)
  out = _sc_lookup(x_flat, table_lin, bias, jnp.asarray(_OFF_EXP))
  return out.reshape(_B, _C, _D)


# speedup vs baseline: 6.0962x; 1.1117x over previous
import functools
import numpy as np
import jax
import jax.numpy as jnp
from jax import lax
from jax.experimental import pallas as pl
from jax.experimental.pallas import tpu as pltpu
from jax.experimental.pallas import tpu_sc as plsc

_C = 26
_D = 32
_B = 16384
_CARD = 100000
_NW = 32
_TOTAL = _B * _C
_PER_W = _TOTAL // _NW
_CHUNK = 832
_NCHUNK = _PER_W // _CHUNK
_L = 16

_NROWS = _C * _CARD + 1
_BJ = 1024
_STRIPE = 650240
_GRID = _STRIPE // _BJ
_ROWS_PAD = 4 * _STRIPE

_OFF_EXP = np.asarray(
    [(p % _C) * _CARD for p in range(208)], dtype=np.int32)


def _body(x_ref, tab_ref, bias_ref, off_ref, out_ref,
          idx_v, off_v, bias_v, rows0, rows1,
          gsem0, gsem1, osem0, osem1):
  cid = lax.axis_index("c")
  sid = lax.axis_index("s")
  wid = sid * 2 + cid
  base = wid * _PER_W

  pltpu.sync_copy(x_ref.at[pl.ds(base, _PER_W)], idx_v)
  pltpu.sync_copy(bias_ref, bias_v)
  pltpu.sync_copy(off_ref, off_v)

  def offs_body(k, carry):
    s = k * _L
    o = off_v[pl.ds(lax.rem(k, 13) * _L, _L)]
    r = idx_v[pl.ds(s, _L)] + o
    q = lax.div(r, _STRIPE)
    idx_v[pl.ds(s, _L)] = 4 * r - q * (4 * _STRIPE - 1)
    return carry
  lax.fori_loop(0, _PER_W // _L, offs_body, 0)

  rows_bufs = (rows0, rows1)
  gsems = (gsem0, gsem1)
  osems = (osem0, osem1)
  ghandles = [None, None]
  ohandles = [None, None]

  def start_gather(g):
    cur = g % 2
    ghandles[cur] = pltpu.async_copy(
        tab_ref.at[idx_v.at[pl.ds(g * _CHUNK, _CHUNK)]],
        rows_bufs[cur], gsems[cur])

  def bias_add(rv):
    def c_body(c, carry):
      b0 = bias_v[c, pl.ds(0, _L)]
      b1 = bias_v[c, pl.ds(_L, _L)]
      def m_body(m, carry2):
        r = m * _C + c
        rv[r, pl.ds(0, _L)] = rv[r, pl.ds(0, _L)] + b0
        rv[r, pl.ds(_L, _L)] = rv[r, pl.ds(_L, _L)] + b1
        return carry2
      return lax.fori_loop(0, _CHUNK // _C, m_body, carry)
    lax.fori_loop(0, _C, c_body, 0)

  start_gather(0)
  for g in range(_NCHUNK):
    cur = g % 2
    nxt = 1 - cur
    if g + 1 < _NCHUNK:
      if ohandles[nxt] is not None:
        ohandles[nxt].wait()
        ohandles[nxt] = None
      start_gather(g + 1)
    ghandles[cur].wait()
    bias_add(rows_bufs[cur])
    ohandles[cur] = pltpu.async_copy(
        rows_bufs[cur],
        out_ref.at[pl.ds(base + g * _CHUNK, _CHUNK)],
        osems[cur])
  for h in ohandles:
    if h is not None:
      h.wait()


@functools.partial(
    pl.kernel,
    out_type=jax.ShapeDtypeStruct((_TOTAL, _D), jnp.float32),
    mesh=plsc.VectorSubcoreMesh(core_axis_name="c", subcore_axis_name="s"),
    compiler_params=pltpu.CompilerParams(use_tc_tiling_on_sc=False),
    scratch_types=[
        pltpu.VMEM((_PER_W,), jnp.int32),
        pltpu.VMEM((len(_OFF_EXP),), jnp.int32),
        pltpu.VMEM((_C, _D), jnp.float32),
        pltpu.VMEM((_CHUNK, _D), jnp.float32),
        pltpu.VMEM((_CHUNK, _D), jnp.float32),
        pltpu.SemaphoreType.DMA,
        pltpu.SemaphoreType.DMA,
        pltpu.SemaphoreType.DMA,
        pltpu.SemaphoreType.DMA,
    ],
)
def _sc_lookup(x_ref, tab_ref, bias_ref, off_ref, out_ref, *scratch):
  _body(x_ref, tab_ref, bias_ref, off_ref, out_ref, *scratch)


def _convert_body(x0, x1, x2, x3, i_ref, o_ref):
  x = jnp.concatenate([r[...] for r in (x0, x1, x2, x3)], axis=0)
  o_ref[...] = jax.lax.dot_general(
      x, i_ref[...], (((0,), (0,)), ((), ())),
      preferred_element_type=jnp.float32,
      precision=jax.lax.Precision.HIGHEST)


def _tc_convert(tab_t, eye):
  return pl.pallas_call(
      _convert_body,
      grid=(_GRID,),
      in_specs=[
          pl.BlockSpec((_D, _BJ), (lambda g, k=k: (0, k * _GRID + g)))
          for k in range(4)
      ] + [pl.BlockSpec((4 * _D, 4 * _D), lambda g: (0, 0))],
      out_specs=pl.BlockSpec((_BJ, 4 * _D), lambda g: (g, 0)),
      out_shape=jax.ShapeDtypeStruct((_STRIPE, 4 * _D), jnp.float32),
  )(tab_t, tab_t, tab_t, tab_t, eye)


def kernel(x_cat, table, bias):
  x_flat = x_cat.astype(jnp.int32).reshape(-1)
  tab_t = jnp.transpose(table)
  conv = _tc_convert(tab_t, jnp.eye(4 * _D, dtype=jnp.float32))
  table_lin = conv.reshape(-1).reshape(_ROWS_PAD, _D)
  out = _sc_lookup(x_flat, table_lin, bias, jnp.asarray(_OFF_EXP))
  return out.reshape(_B, _C, _D)


# speedup vs baseline: 7.2884x; 1.1956x over previous
import functools
import numpy as np
import jax
import jax.numpy as jnp
from jax import lax
from jax.experimental import pallas as pl
from jax.experimental.pallas import tpu as pltpu
from jax.experimental.pallas import tpu_sc as plsc

_C = 26
_D = 32
_B = 16384
_CARD = 100000
_NW = 32
_TOTAL = _B * _C
_PER_W = _TOTAL // _NW
_CHUNK = 832
_NCHUNK = _PER_W // _CHUNK
_L = 16

_NROWS = _C * _CARD + 1
_BJ = 2048
_STRIPE = 651264
_GRID = _STRIPE // _BJ
_MAXBLK = (_NROWS - 1) // _BJ
_ROWS_PAD = 4 * _STRIPE

_OFF_EXP = np.asarray(
    [(p % _C) * _CARD for p in range(208)], dtype=np.int32)


def _body(x_ref, tab_ref, bias_ref, off_ref, out_ref,
          idx_v, off_v, bias_v, rows0, rows1,
          gsem0, gsem1, osem0, osem1):
  cid = lax.axis_index("c")
  sid = lax.axis_index("s")
  wid = sid * 2 + cid
  base = wid * _PER_W

  pltpu.sync_copy(x_ref.at[pl.ds(base, _PER_W)], idx_v)
  pltpu.sync_copy(bias_ref, bias_v)
  pltpu.sync_copy(off_ref, off_v)

  def offs_body(k, carry):
    s = k * _L
    o = off_v[pl.ds(lax.rem(k, 13) * _L, _L)]
    r = idx_v[pl.ds(s, _L)] + o
    q = lax.div(r, _STRIPE)
    idx_v[pl.ds(s, _L)] = 4 * r - q * (4 * _STRIPE - 1)
    return carry
  lax.fori_loop(0, _PER_W // _L, offs_body, 0)

  rows_bufs = (rows0, rows1)
  gsems = (gsem0, gsem1)
  osems = (osem0, osem1)
  ghandles = [None, None]
  ohandles = [None, None]

  def start_gather(g):
    cur = g % 2
    ghandles[cur] = pltpu.async_copy(
        tab_ref.at[idx_v.at[pl.ds(g * _CHUNK, _CHUNK)]],
        rows_bufs[cur], gsems[cur])

  def bias_add(rv):
    def c_body(c, carry):
      b0 = bias_v[c, pl.ds(0, _L)]
      b1 = bias_v[c, pl.ds(_L, _L)]
      def m_body(m, carry2):
        r = m * _C + c
        rv[r, pl.ds(0, _L)] = rv[r, pl.ds(0, _L)] + b0
        rv[r, pl.ds(_L, _L)] = rv[r, pl.ds(_L, _L)] + b1
        return carry2
      return lax.fori_loop(0, _CHUNK // _C, m_body, carry)
    lax.fori_loop(0, _C, c_body, 0)

  start_gather(0)
  for g in range(_NCHUNK):
    cur = g % 2
    nxt = 1 - cur
    if g + 1 < _NCHUNK:
      if ohandles[nxt] is not None:
        ohandles[nxt].wait()
        ohandles[nxt] = None
      start_gather(g + 1)
    ghandles[cur].wait()
    bias_add(rows_bufs[cur])
    ohandles[cur] = pltpu.async_copy(
        rows_bufs[cur],
        out_ref.at[pl.ds(base + g * _CHUNK, _CHUNK)],
        osems[cur])
  for h in ohandles:
    if h is not None:
      h.wait()


@functools.partial(
    pl.kernel,
    out_type=jax.ShapeDtypeStruct((_TOTAL, _D), jnp.float32),
    mesh=plsc.VectorSubcoreMesh(core_axis_name="c", subcore_axis_name="s"),
    compiler_params=pltpu.CompilerParams(use_tc_tiling_on_sc=False),
    scratch_types=[
        pltpu.VMEM((_PER_W,), jnp.int32),
        pltpu.VMEM((len(_OFF_EXP),), jnp.int32),
        pltpu.VMEM((_C, _D), jnp.float32),
        pltpu.VMEM((_CHUNK, _D), jnp.float32),
        pltpu.VMEM((_CHUNK, _D), jnp.float32),
        pltpu.SemaphoreType.DMA,
        pltpu.SemaphoreType.DMA,
        pltpu.SemaphoreType.DMA,
        pltpu.SemaphoreType.DMA,
    ],
)
def _sc_lookup(x_ref, tab_ref, bias_ref, off_ref, out_ref, *scratch):
  _body(x_ref, tab_ref, bias_ref, off_ref, out_ref, *scratch)


def _convert_body(x0, x1, x2, x3, i_ref, o_ref):
  x = jnp.concatenate([r[...] for r in (x0, x1, x2, x3)], axis=0)
  o_ref[...] = jax.lax.dot_general(
      x, i_ref[...], (((0,), (0,)), ((), ())),
      preferred_element_type=jnp.float32,
      precision=jax.lax.Precision.HIGHEST)


def _tc_convert(tab_t, eye):
  return pl.pallas_call(
      _convert_body,
      grid=(_GRID,),
      in_specs=[
          pl.BlockSpec(
              (_D, _BJ),
              (lambda g, k=k: (0, jnp.minimum(k * _GRID + g, _MAXBLK))))
          for k in range(4)
      ] + [pl.BlockSpec((4 * _D, 4 * _D), lambda g: (0, 0))],
      out_specs=pl.BlockSpec((_BJ, 4 * _D), lambda g: (g, 0)),
      out_shape=jax.ShapeDtypeStruct((_STRIPE, 4 * _D), jnp.float32),
  )(tab_t, tab_t, tab_t, tab_t, eye)


def kernel(x_cat, table, bias):
  x_flat = x_cat.astype(jnp.int32).reshape(-1)
  tab_t = jnp.transpose(table)
  conv = _tc_convert(tab_t, jnp.eye(4 * _D, dtype=jnp.float32))
  table_lin = conv.reshape(-1).reshape(_ROWS_PAD, _D)
  out = _sc_lookup(x_flat, table_lin, bias, jnp.asarray(_OFF_EXP))
  return out.reshape(_B, _C, _D)
